# Initial kernel scaffold; baseline (speedup 1.0000x reference)
import jax
import jax.numpy as jnp
from jax.experimental import pallas as pl


def kernel(in_coords, in_feats, grid_points, W, gamma, beta):
    raise NotImplementedError("write your pallas kernel here")


# speedup vs baseline: 7.7907x; 7.7907x over previous
import functools

import jax
import jax.numpy as jnp
import numpy as np
from jax.experimental import pallas as pl
from jax.experimental.pallas import tpu as pltpu

N_IN = 100000
G = 55296
C = 128
VOX = np.float32(0.04)
T_IN = 2 * 130 * 130 * 130
T_GV = 2 * 128 * 128 * 128
GB = 512
NGB = G // GB
KK = 27


def _conv_stats_kernel(f_ref, w_ref, rep_ref, out_ref, s1_ref, s2_ref):
    gb = pl.program_id(0)
    k = pl.program_id(1)

    @pl.when(k == 0)
    def _():
        out_ref[...] = jnp.zeros_like(out_ref)

    @pl.when(jnp.logical_and(gb == 0, k == 0))
    def _():
        s1_ref[...] = jnp.zeros_like(s1_ref)
        s2_ref[...] = jnp.zeros_like(s2_ref)

    out_ref[...] += jnp.dot(f_ref[0], w_ref[0],
                            preferred_element_type=jnp.float32)

    @pl.when(k == KK - 1)
    def _():
        o = out_ref[...]
        r = rep_ref[...]
        s1_ref[...] += jnp.sum(o * r, axis=0, keepdims=True)
        s2_ref[...] += jnp.sum(o * o * r, axis=0, keepdims=True)


def _norm_elu_kernel(o_ref, sc_ref, sh_ref, y_ref):
    x = o_ref[...] * sc_ref[...] + sh_ref[...]
    y_ref[...] = jnp.where(x > 0, x, jnp.exp(jnp.minimum(x, 0.0)) - 1.0)


def kernel(in_coords, in_feats, grid_points, W, gamma, beta):
    ic = in_coords.astype(jnp.int32)
    b_in = ic[:, 0]
    h = ic[:, 1:4] // 2
    ikey = ((b_in * 130 + h[:, 0] + 65) * 130 + h[:, 1] + 65) * 130 + h[:, 2] + 65
    in_table = (jnp.full((T_IN,), N_IN, jnp.int32)
                .at[ikey].min(jnp.arange(N_IN, dtype=jnp.int32)))

    sp = jnp.floor(grid_points[:, 1:4] / VOX).astype(jnp.int32)
    bg = grid_points[:, 0].astype(jnp.int32)
    qbase = ((bg * 130 + sp[:, 0] + 65) * 130 + sp[:, 1] + 65) * 130 + sp[:, 2] + 65
    gvox = ((bg * 128 + sp[:, 0] + 64) * 128 + sp[:, 1] + 64) * 128 + sp[:, 2] + 64
    gt = (jnp.full((T_GV,), G, jnp.int32)
          .at[gvox].min(jnp.arange(G, dtype=jnp.int32)))
    rep = (gt[gvox] == jnp.arange(G, dtype=jnp.int32)).astype(jnp.float32)

    doff = jnp.array([dx * 16900 + dy * 130 + dz
                      for dx in (-1, 0, 1) for dy in (-1, 0, 1)
                      for dz in (-1, 0, 1)], jnp.int32)
    q = qbase[None, :] + doff[:, None]
    j = in_table[q]
    feats_pad = jnp.concatenate(
        [in_feats, jnp.zeros((1, C), jnp.float32)], axis=0)
    F = feats_pad[j]

    out, s1, s2 = pl.pallas_call(
        _conv_stats_kernel,
        grid=(NGB, KK),
        in_specs=[
            pl.BlockSpec((1, GB, C), lambda gb, k: (k, gb, k * 0)),
            pl.BlockSpec((1, C, C), lambda gb, k: (k, k * 0, k * 0)),
            pl.BlockSpec((GB, 1), lambda gb, k: (gb, k * 0)),
        ],
        out_specs=[
            pl.BlockSpec((GB, C), lambda gb, k: (gb, k * 0)),
            pl.BlockSpec((1, C), lambda gb, k: (k * 0, k * 0)),
            pl.BlockSpec((1, C), lambda gb, k: (k * 0, k * 0)),
        ],
        out_shape=[
            jax.ShapeDtypeStruct((G, C), jnp.float32),
            jax.ShapeDtypeStruct((1, C), jnp.float32),
            jax.ShapeDtypeStruct((1, C), jnp.float32),
        ],
    )(F, W.astype(jnp.float32), rep[:, None])

    n = jnp.sum(rep)
    mean = s1[0] / n
    var = s2[0] / n - mean * mean
    scale = gamma / jnp.sqrt(var + 1e-5)
    shift = beta - mean * scale

    y = pl.pallas_call(
        _norm_elu_kernel,
        grid=(NGB,),
        in_specs=[
            pl.BlockSpec((GB, C), lambda gb: (gb, gb * 0)),
            pl.BlockSpec((1, C), lambda gb: (gb * 0, gb * 0)),
            pl.BlockSpec((1, C), lambda gb: (gb * 0, gb * 0)),
        ],
        out_specs=pl.BlockSpec((GB, C), lambda gb: (gb, gb * 0)),
        out_shape=jax.ShapeDtypeStruct((G, C), jnp.float32),
    )(out, scale[None, :], shift[None, :])
    return y
